# baseline (speedup 1.0000x reference)
import functools

import jax
import jax.numpy as jnp
from jax import lax
from jax.experimental import pallas as pl
from jax.experimental.pallas import tpu as pltpu
from jax.experimental.pallas import tpu_sc as plsc

_TM = 128
_TILE_R = 1024


def _route_body(x_ref, wr_ref, br_ref, estar_ref, w16_ref):
    xt = x_ref[...]
    logits = lax.dot_general(xt, wr_ref[...], (((1,), (1,)), ((), ())),
                             preferred_element_type=jnp.float32)
    logits = logits + br_ref[...]
    max1 = jnp.max(logits, axis=1, keepdims=True)
    ex = jnp.exp(logits - max1)
    probs = ex / jnp.sum(ex, axis=1, keepdims=True)
    iota_e = lax.broadcasted_iota(jnp.int32, logits.shape, 1)
    big = jnp.asarray(logits.shape[1], jnp.int32)
    e1 = jnp.min(jnp.where(logits == max1, iota_e, big), axis=1, keepdims=True)
    l2 = jnp.where(iota_e == e1, -jnp.inf, logits)
    max2 = jnp.max(l2, axis=1, keepdims=True)
    e2 = jnp.min(jnp.where(l2 == max2, iota_e, big), axis=1, keepdims=True)
    es = jnp.maximum(e1, e2)
    w = jnp.sum(jnp.where(iota_e == es, probs, 0.0), axis=1, keepdims=True)
    estar_ref[...] = es
    w16_ref[...] = jnp.broadcast_to(w, (w.shape[0], 128))


def _rank_flat(m, lane, sub):
    rows, lanes = m.shape
    c = m
    k = 1
    while k < lanes:
        c = c + jnp.where(lane >= k, pltpu.roll(c, k, 1), 0)
        k *= 2
    rowtot = c[:, lanes - 1:lanes]
    r = rowtot
    k = 1
    while k < rows:
        r = r + jnp.where(sub >= k, pltpu.roll(r, k, 0), 0)
        k *= 2
    return (c - m) + (r - rowtot)


def _sort_half(est, num_e, tm):
    rows, lanes = est.shape
    lane = lax.broadcasted_iota(jnp.int32, (rows, lanes), 1)
    sub = lax.broadcasted_iota(jnp.int32, (rows, 1), 0)
    pos = jnp.zeros((rows, lanes), jnp.int32)
    offs = []
    off = jnp.asarray(0, jnp.int32)
    for e in range(num_e):
        m = (est == e).astype(jnp.int32)
        excl = _rank_flat(m, lane, sub)
        pos = pos + jnp.where(m, excl + off, 0)
        offs.append(off)
        cnt = jnp.sum(m)
        off = off + ((cnt + (tm - 1)) // tm) * tm
    bi = (lax.broadcasted_iota(jnp.int32, (8, 128), 0) * 128
          + lax.broadcasted_iota(jnp.int32, (8, 128), 1))
    te = jnp.zeros((8, 128), jnp.int32)
    for e in range(num_e):
        te = te + jnp.where(bi * tm >= offs[e], 1, 0)
    return pos, te - 1


def _sort_body(est_ref, pos_ref, tea_ref, teb_ref, num_e, tm):
    est = est_ref[...]
    rows = est.shape[0]
    h = rows // 2
    pos_a, te_a = _sort_half(est[:h], num_e, tm)
    pos_b, te_b = _sort_half(est[h:], num_e, tm)
    pos_ref[...] = jnp.concatenate([pos_a, pos_b], axis=0)
    tea_ref[...] = te_a
    teb_ref[...] = te_b


def _make_dispatch(nh, pad_n, d, nw, chunk, tok_base):
    rows_w = nh // nw
    nch = rows_w // chunk
    mesh = plsc.VectorSubcoreMesh(core_axis_name="c", subcore_axis_name="s")

    @functools.partial(
        pl.kernel, mesh=mesh,
        out_type=(jax.ShapeDtypeStruct((pad_n, d), jnp.float32),
                  jax.ShapeDtypeStruct((pad_n, 128), jnp.float32)),
        scratch_types=[pltpu.VMEM((nch, chunk), jnp.int32),
                       pltpu.VMEM((chunk, d), jnp.float32),
                       pltpu.VMEM((chunk, d), jnp.float32),
                       pltpu.VMEM((chunk, 128), jnp.float32),
                       pltpu.VMEM((chunk, 128), jnp.float32),
                       pltpu.SemaphoreType.DMA,
                       pltpu.SemaphoreType.DMA,
                       pltpu.SemaphoreType.DMA,
                       pltpu.SemaphoreType.DMA],
    )
    def dispatch(x_hbm, w16_hbm, pos_hbm, xs_hbm, ws_hbm,
                 idx_v, xbuf0, xbuf1, wbuf0, wbuf1, semxi, semwi, semxo, semwo):
        cid = lax.axis_index("c")
        sid = lax.axis_index("s")
        wid = sid * 2 + cid
        base = tok_base + wid * rows_w
        xbufs = (xbuf0, xbuf1)
        wbufs = (wbuf0, wbuf1)
        for k in range(nch):
            pltpu.sync_copy(pos_hbm.at[pl.ds(base + k * chunk, chunk)],
                            idx_v.at[k])
        pltpu.sync_copy(x_hbm.at[pl.ds(base, chunk)], xbufs[0])
        pltpu.sync_copy(w16_hbm.at[pl.ds(base, chunk)], wbufs[0])
        for k in range(nch):
            kb = k % 2
            if k + 1 < nch:
                cpx = pltpu.async_copy(
                    x_hbm.at[pl.ds(base + (k + 1) * chunk, chunk)],
                    xbufs[1 - kb], semxi)
                cpw = pltpu.async_copy(
                    w16_hbm.at[pl.ds(base + (k + 1) * chunk, chunk)],
                    wbufs[1 - kb], semwi)
            sox = pltpu.async_copy(xbufs[kb], xs_hbm.at[idx_v.at[k]], semxo)
            sow = pltpu.async_copy(wbufs[kb], ws_hbm.at[idx_v.at[k]], semwo)
            sox.wait()
            sow.wait()
            if k + 1 < nch:
                cpx.wait()
                cpw.wait()

    return dispatch


def _gmm_body(te_ref, xs_ref, ws_ref, we_ref, be_ref, ys_ref):
    del te_ref
    acc = lax.dot_general(xs_ref[...], we_ref[0], (((1,), (1,)), ((), ())),
                          preferred_element_type=jnp.float32)
    ys_ref[...] = (acc + be_ref[0]) * ws_ref[:, 0:1]


def _make_collect(nh, pad_n, d, nw, chunk, tok_base):
    rows_w = nh // nw
    nch = rows_w // chunk
    mesh = plsc.VectorSubcoreMesh(core_axis_name="c", subcore_axis_name="s")

    @functools.partial(
        pl.kernel, mesh=mesh,
        out_type=jax.ShapeDtypeStruct((nh, d), jnp.float32),
        scratch_types=[pltpu.VMEM((rows_w,), jnp.int32),
                       pltpu.VMEM((chunk, d), jnp.float32),
                       pltpu.VMEM((chunk, d), jnp.float32),
                       pltpu.SemaphoreType.DMA,
                       pltpu.SemaphoreType.DMA],
    )
    def collect(ys_hbm, pos_hbm, out_hbm, idx_v, buf0, buf1, semi, semo):
        cid = lax.axis_index("c")
        sid = lax.axis_index("s")
        wid = sid * 2 + cid
        base = wid * rows_w
        bufs = (buf0, buf1)
        pltpu.sync_copy(pos_hbm.at[pl.ds(tok_base + base, rows_w)], idx_v)
        pltpu.async_copy(
            ys_hbm.at[idx_v.at[pl.ds(0, chunk)]], bufs[0], semi).wait()
        for k in range(nch):
            kb = k % 2
            if k + 1 < nch:
                cp = pltpu.async_copy(
                    ys_hbm.at[idx_v.at[pl.ds((k + 1) * chunk, chunk)]],
                    bufs[1 - kb], semi)
            so = pltpu.async_copy(
                bufs[kb], out_hbm.at[pl.ds(base + k * chunk, chunk)], semo)
            so.wait()
            if k + 1 < nch:
                cp.wait()

    return collect


def _gmm(te_flat, xs, ws, We, be3, g, tm, d):
    grid_spec = pltpu.PrefetchScalarGridSpec(
        num_scalar_prefetch=1,
        grid=(g,),
        in_specs=[
            pl.BlockSpec((tm, d), lambda g, te: (g, 0)),
            pl.BlockSpec((tm, 128), lambda g, te: (g, 0)),
            pl.BlockSpec((1, d, d), lambda g, te: (te[g], 0, 0)),
            pl.BlockSpec((1, 1, d), lambda g, te: (te[g], 0, 0)),
        ],
        out_specs=pl.BlockSpec((tm, d), lambda g, te: (g, 0)),
    )
    return pl.pallas_call(
        _gmm_body,
        grid_spec=grid_spec,
        out_shape=jax.ShapeDtypeStruct((xs.shape[0], d), jnp.float32),
        compiler_params=pltpu.CompilerParams(
            dimension_semantics=("arbitrary",)),
    )(te_flat, xs, ws, We, be3)


def kernel(x, Wr, br, We, be):
    B, S, D = x.shape
    E = Wr.shape[0]
    N = B * S
    NH = N // 2
    PAD_H = NH + E * _TM
    GH = PAD_H // _TM
    NW = 32

    x2 = x.reshape(N, D)
    br2 = br.reshape(1, E)
    be3 = be.reshape(E, 1, D)

    estar, w16 = pl.pallas_call(
        _route_body,
        grid=(N // _TILE_R,),
        in_specs=[
            pl.BlockSpec((_TILE_R, D), lambda m: (m, 0)),
            pl.BlockSpec((E, D), lambda m: (0, 0)),
            pl.BlockSpec((1, E), lambda m: (0, 0)),
        ],
        out_specs=[
            pl.BlockSpec((_TILE_R, 1), lambda m: (m, 0)),
            pl.BlockSpec((_TILE_R, 128), lambda m: (m, 0)),
        ],
        out_shape=[
            jax.ShapeDtypeStruct((N, 1), jnp.int32),
            jax.ShapeDtypeStruct((N, 128), jnp.float32),
        ],
        compiler_params=pltpu.CompilerParams(
            dimension_semantics=("parallel",)),
    )(x2, Wr, br2)

    est64 = estar.reshape(N // 128, 128)
    pos64, tea, teb = pl.pallas_call(
        functools.partial(_sort_body, num_e=E, tm=_TM),
        out_shape=[
            jax.ShapeDtypeStruct((N // 128, 128), jnp.int32),
            jax.ShapeDtypeStruct((8, 128), jnp.int32),
            jax.ShapeDtypeStruct((8, 128), jnp.int32),
        ],
    )(est64)
    pos_flat = pos64.reshape(N)
    tea_flat = tea.reshape(-1)[:GH]
    teb_flat = teb.reshape(-1)[:GH]

    xs_a, ws_a = _make_dispatch(NH, PAD_H, D, NW, 32, 0)(x2, w16, pos_flat)
    xs_b, ws_b = _make_dispatch(NH, PAD_H, D, NW, 32, NH)(x2, w16, pos_flat)
    ys_a = _gmm(tea_flat, xs_a, ws_a, We, be3, GH, _TM, D)
    ys_b = _gmm(teb_flat, xs_b, ws_b, We, be3, GH, _TM, D)
    out_a = _make_collect(NH, PAD_H, D, NW, 32, 0)(ys_a, pos_flat)
    out_b = _make_collect(NH, PAD_H, D, NW, 32, NH)(ys_b, pos_flat)
    out2 = jnp.concatenate([out_a, out_b], axis=0)
    return out2.reshape(B, S, D)


# speedup vs baseline: 1.4752x; 1.4752x over previous
import functools

import jax
import jax.numpy as jnp
from jax import lax
from jax.experimental import pallas as pl
from jax.experimental.pallas import tpu as pltpu
from jax.experimental.pallas import tpu_sc as plsc

_TM = 512
_TILE_R = 1024


def _route_body(x_ref, wr_ref, br_ref, w16_ref, pos_ref, te_ref, est_scr,
                num_e, tm):
    m_id = pl.program_id(0)
    xt = x_ref[...]
    logits = lax.dot_general(xt, wr_ref[...], (((1,), (1,)), ((), ())),
                             preferred_element_type=jnp.float32)
    logits = logits + br_ref[...]
    max1 = jnp.max(logits, axis=1, keepdims=True)
    ex = jnp.exp(logits - max1)
    probs = ex / jnp.sum(ex, axis=1, keepdims=True)
    iota_e = lax.broadcasted_iota(jnp.int32, logits.shape, 1)
    big = jnp.asarray(logits.shape[1], jnp.int32)
    e1 = jnp.min(jnp.where(logits == max1, iota_e, big), axis=1, keepdims=True)
    l2 = jnp.where(iota_e == e1, -jnp.inf, logits)
    max2 = jnp.max(l2, axis=1, keepdims=True)
    e2 = jnp.min(jnp.where(l2 == max2, iota_e, big), axis=1, keepdims=True)
    es = jnp.maximum(e1, e2)
    w = jnp.sum(jnp.where(iota_e == es, probs, 0.0), axis=1, keepdims=True)
    rows_t = xt.shape[0] // 128
    est_scr[pl.ds(m_id * rows_t, rows_t), :] = es.reshape(rows_t, 128)
    w16_ref[...] = jnp.broadcast_to(w, (w.shape[0], 128))

    @pl.when(m_id == pl.num_programs(0) - 1)
    def _sort():
        pos, te = _sort_half(est_scr[...], num_e, tm)
        pos_ref[...] = pos
        te_ref[...] = te


def _rank_flat(m, lane, sub):
    rows, lanes = m.shape
    c = m
    k = 1
    while k < lanes:
        c = c + jnp.where(lane >= k, pltpu.roll(c, k, 1), 0)
        k *= 2
    rowtot = c[:, lanes - 1:lanes]
    r = rowtot
    k = 1
    while k < rows:
        r = r + jnp.where(sub >= k, pltpu.roll(r, k, 0), 0)
        k *= 2
    return (c - m) + (r - rowtot)


def _sort_half(est, num_e, tm):
    rows, lanes = est.shape
    lane = lax.broadcasted_iota(jnp.int32, (rows, lanes), 1)
    sub = lax.broadcasted_iota(jnp.int32, (rows, 1), 0)
    pos = jnp.zeros((rows, lanes), jnp.int32)
    offs = []
    cnts = []
    off = jnp.asarray(0, jnp.int32)
    for e in range(num_e):
        m = (est == e).astype(jnp.int32)
        excl = _rank_flat(m, lane, sub)
        pos = pos + jnp.where(m, excl + off, 0)
        offs.append(off)
        cnt = jnp.sum(m)
        cnts.append(cnt)
        off = off + ((cnt + (tm - 1)) // tm) * tm
    bi = (lax.broadcasted_iota(jnp.int32, (8, 128), 0) * 128
          + lax.broadcasted_iota(jnp.int32, (8, 128), 1))
    te = jnp.zeros((8, 128), jnp.int32)
    act = jnp.zeros((8, 128), jnp.int32)
    for e in range(num_e):
        te = te + jnp.where(bi * tm >= offs[e], 1, 0)
        act = act | ((bi * tm >= offs[e]) & (bi * tm < offs[e] + cnts[e]))
    return pos, (te - 1) + 8 * (1 - act)


def _make_dispatch(n, pad_n, d, nw, chunk):
    rows_w = n // nw
    nch = rows_w // chunk
    mesh = plsc.VectorSubcoreMesh(core_axis_name="c", subcore_axis_name="s")

    @functools.partial(
        pl.kernel, mesh=mesh,
        out_type=(jax.ShapeDtypeStruct((pad_n, d), jnp.float32),
                  jax.ShapeDtypeStruct((pad_n, 128), jnp.float32)),
        scratch_types=[pltpu.VMEM((nch, chunk), jnp.int32),
                       pltpu.VMEM((chunk, d), jnp.float32),
                       pltpu.VMEM((chunk, 128), jnp.float32),
                       pltpu.SemaphoreType.DMA,
                       pltpu.SemaphoreType.DMA],
    )
    def dispatch(x_hbm, w16_hbm, pos_hbm, xs_hbm, ws_hbm,
                 idx_v, xbuf, wbuf, semx, semw):
        cid = lax.axis_index("c")
        sid = lax.axis_index("s")
        wid = sid * 2 + cid
        base = wid * rows_w
        for k in range(nch):
            pltpu.sync_copy(pos_hbm.at[pl.ds(base + k * chunk, chunk)],
                            idx_v.at[k])
        for k in range(nch):
            pltpu.sync_copy(x_hbm.at[pl.ds(base + k * chunk, chunk)], xbuf)
            pltpu.sync_copy(w16_hbm.at[pl.ds(base + k * chunk, chunk)], wbuf)
            cpx = pltpu.async_copy(xbuf, xs_hbm.at[idx_v.at[k]], semx)
            cpw = pltpu.async_copy(wbuf, ws_hbm.at[idx_v.at[k]], semw)
            cpx.wait()
            cpw.wait()

    return dispatch


def _gmm_body(te_ref, xs_ref, ws_ref, we_ref, be_ref, ys_ref):
    v = te_ref[pl.program_id(0)]

    @pl.when(v < 8)
    def _():
        e = v
        we = we_ref[pl.ds(e, 1), :, :][0]
        acc = lax.dot_general(xs_ref[...], we, (((1,), (1,)), ((), ())),
                              preferred_element_type=jnp.float32)
        ys_ref[...] = (acc + be_ref[pl.ds(e, 1), 0, :]) * ws_ref[:, 0:1]


def _make_collect(n, pad_n, d, nw, chunk):
    rows_w = n // nw
    nch = rows_w // chunk
    mesh = plsc.VectorSubcoreMesh(core_axis_name="c", subcore_axis_name="s")

    del chunk
    ch = 56
    sizes = []
    left = rows_w
    while left > 0:
        sizes.append(min(ch, left))
        left -= min(ch, left)

    @functools.partial(
        pl.kernel, mesh=mesh,
        out_type=jax.ShapeDtypeStruct((n, d), jnp.float32),
        scratch_types=[pltpu.VMEM((rows_w,), jnp.int32),
                       pltpu.VMEM((ch, d), jnp.float32),
                       pltpu.VMEM((ch, d), jnp.float32),
                       pltpu.SemaphoreType.DMA,
                       pltpu.SemaphoreType.DMA],
    )
    def collect(ys_hbm, pos_hbm, out_hbm, idx_v, buf0, buf1, semi, semo):
        cid = lax.axis_index("c")
        sid = lax.axis_index("s")
        wid = sid * 2 + cid
        base = wid * rows_w
        bufs = (buf0, buf1)
        offs = [sum(sizes[:k]) for k in range(len(sizes))]
        pltpu.sync_copy(pos_hbm.at[pl.ds(base, rows_w)], idx_v)
        pltpu.async_copy(
            ys_hbm.at[idx_v.at[pl.ds(0, sizes[0])]],
            bufs[0].at[pl.ds(0, sizes[0])], semi).wait()
        for k in range(len(sizes)):
            kb = k % 2
            if k + 1 < len(sizes):
                cp = pltpu.async_copy(
                    ys_hbm.at[idx_v.at[pl.ds(offs[k + 1], sizes[k + 1])]],
                    bufs[1 - kb].at[pl.ds(0, sizes[k + 1])], semi)
            so = pltpu.async_copy(
                bufs[kb].at[pl.ds(0, sizes[k])],
                out_hbm.at[pl.ds(base + offs[k], sizes[k])], semo)
            so.wait()
            if k + 1 < len(sizes):
                cp.wait()

    return collect


def _gmm(te_flat, xs, ws, We, be3, g, tm, d):
    grid_spec = pltpu.PrefetchScalarGridSpec(
        num_scalar_prefetch=1,
        grid=(g,),
        in_specs=[
            pl.BlockSpec((tm, d), lambda g, te: (g, 0)),
            pl.BlockSpec((tm, 128), lambda g, te: (g, 0)),
            pl.BlockSpec((8, d, d), lambda g, te: (0, 0, 0)),
            pl.BlockSpec((8, 1, d), lambda g, te: (0, 0, 0)),
        ],
        out_specs=pl.BlockSpec((tm, d), lambda g, te: (g, 0)),
    )
    return pl.pallas_call(
        _gmm_body,
        grid_spec=grid_spec,
        out_shape=jax.ShapeDtypeStruct((xs.shape[0], d), jnp.float32),
        compiler_params=pltpu.CompilerParams(
            dimension_semantics=("arbitrary",)),
    )(te_flat, xs, ws, We, be3)


def kernel(x, Wr, br, We, be):
    B, S, D = x.shape
    E = Wr.shape[0]
    N = B * S
    PAD_N = N + E * _TM
    G = PAD_N // _TM
    NW = 32

    x2 = x.reshape(N, D)
    br2 = br.reshape(1, E)
    be3 = be.reshape(E, 1, D)

    w16, pos64, te8 = pl.pallas_call(
        functools.partial(_route_body, num_e=E, tm=_TM),
        grid=(N // _TILE_R,),
        in_specs=[
            pl.BlockSpec((_TILE_R, D), lambda m: (m, 0)),
            pl.BlockSpec((E, D), lambda m: (0, 0)),
            pl.BlockSpec((1, E), lambda m: (0, 0)),
        ],
        out_specs=[
            pl.BlockSpec((_TILE_R, 128), lambda m: (m, 0)),
            pl.BlockSpec((N // 128, 128), lambda m: (0, 0)),
            pl.BlockSpec((8, 128), lambda m: (0, 0)),
        ],
        out_shape=[
            jax.ShapeDtypeStruct((N, 128), jnp.float32),
            jax.ShapeDtypeStruct((N // 128, 128), jnp.int32),
            jax.ShapeDtypeStruct((8, 128), jnp.int32),
        ],
        scratch_shapes=[pltpu.VMEM((N // 128, 128), jnp.int32)],
        compiler_params=pltpu.CompilerParams(
            dimension_semantics=("arbitrary",)),
    )(x2, Wr, br2)
    pos_flat = pos64.reshape(N)
    te_flat = te8.reshape(-1)[:G]

    xs, ws = _make_dispatch(N, PAD_N, D, NW, 64)(x2, w16, pos_flat)
    ys = _gmm(te_flat, xs, ws, We, be3, G, _TM, D)
    out2 = _make_collect(N, PAD_N, D, NW, 64)(ys, pos_flat)
    return out2.reshape(B, S, D)


# speedup vs baseline: 1.4840x; 1.0059x over previous
import functools

import jax
import jax.numpy as jnp
from jax import lax
from jax.experimental import pallas as pl
from jax.experimental.pallas import tpu as pltpu
from jax.experimental.pallas import tpu_sc as plsc

_TM = 512
_TILE_R = 2048


def _route_body(x_ref, wr_ref, br_ref, w16_ref, pos_ref, te_ref, est_scr,
                num_e, tm):
    m_id = pl.program_id(0)
    xt = x_ref[...]
    logits = lax.dot_general(xt, wr_ref[...], (((1,), (1,)), ((), ())),
                             preferred_element_type=jnp.float32)
    logits = logits + br_ref[...]
    max1 = jnp.max(logits, axis=1, keepdims=True)
    ex = jnp.exp(logits - max1)
    probs = ex / jnp.sum(ex, axis=1, keepdims=True)
    iota_e = lax.broadcasted_iota(jnp.int32, logits.shape, 1)
    big = jnp.asarray(logits.shape[1], jnp.int32)
    e1 = jnp.min(jnp.where(logits == max1, iota_e, big), axis=1, keepdims=True)
    l2 = jnp.where(iota_e == e1, -jnp.inf, logits)
    max2 = jnp.max(l2, axis=1, keepdims=True)
    e2 = jnp.min(jnp.where(l2 == max2, iota_e, big), axis=1, keepdims=True)
    es = jnp.maximum(e1, e2)
    w = jnp.sum(jnp.where(iota_e == es, probs, 0.0), axis=1, keepdims=True)
    rows_t = xt.shape[0] // 128
    est_scr[pl.ds(m_id * rows_t, rows_t), :] = es.reshape(rows_t, 128)
    w16_ref[...] = jnp.broadcast_to(w, (w.shape[0], 128))

    @pl.when(m_id == pl.num_programs(0) - 1)
    def _sort():
        pos, te = _sort_half(est_scr[...], num_e, tm)
        pos_ref[...] = pos
        te_ref[...] = te


def _rank_flat(m, lane, sub):
    rows, lanes = m.shape
    c = m
    k = 1
    while k < lanes:
        c = c + jnp.where(lane >= k, pltpu.roll(c, k, 1), 0)
        k *= 2
    rowtot = c[:, lanes - 1:lanes]
    r = rowtot
    k = 1
    while k < rows:
        r = r + jnp.where(sub >= k, pltpu.roll(r, k, 0), 0)
        k *= 2
    return (c - m) + (r - rowtot)


def _sort_half(est, num_e, tm):
    rows, lanes = est.shape
    lane = lax.broadcasted_iota(jnp.int32, (rows, lanes), 1)
    sub = lax.broadcasted_iota(jnp.int32, (rows, 1), 0)
    pos = jnp.zeros((rows, lanes), jnp.int32)
    offs = []
    cnts = []
    off = jnp.asarray(0, jnp.int32)
    for e in range(num_e):
        m = (est == e).astype(jnp.int32)
        excl = _rank_flat(m, lane, sub)
        pos = pos + jnp.where(m, excl + off, 0)
        offs.append(off)
        cnt = jnp.sum(m)
        cnts.append(cnt)
        off = off + ((cnt + (tm - 1)) // tm) * tm
    bi = (lax.broadcasted_iota(jnp.int32, (8, 128), 0) * 128
          + lax.broadcasted_iota(jnp.int32, (8, 128), 1))
    te = jnp.zeros((8, 128), jnp.int32)
    act = jnp.zeros((8, 128), jnp.int32)
    for e in range(num_e):
        te = te + jnp.where(bi * tm >= offs[e], 1, 0)
        act = act | ((bi * tm >= offs[e]) & (bi * tm < offs[e] + cnts[e]))
    return pos, (te - 1) + 8 * (1 - act)


def _make_dispatch(n, pad_n, d, nw, chunk):
    rows_w = n // nw
    nch = rows_w // chunk
    mesh = plsc.VectorSubcoreMesh(core_axis_name="c", subcore_axis_name="s")

    @functools.partial(
        pl.kernel, mesh=mesh,
        out_type=(jax.ShapeDtypeStruct((pad_n, d), jnp.float32),
                  jax.ShapeDtypeStruct((pad_n, 128), jnp.float32)),
        scratch_types=[pltpu.VMEM((nch, chunk), jnp.int32),
                       pltpu.VMEM((chunk, d), jnp.float32),
                       pltpu.VMEM((chunk, 128), jnp.float32),
                       pltpu.SemaphoreType.DMA,
                       pltpu.SemaphoreType.DMA],
    )
    def dispatch(x_hbm, w16_hbm, pos_hbm, xs_hbm, ws_hbm,
                 idx_v, xbuf, wbuf, semx, semw):
        cid = lax.axis_index("c")
        sid = lax.axis_index("s")
        wid = sid * 2 + cid
        base = wid * rows_w
        for k in range(nch):
            pltpu.sync_copy(pos_hbm.at[pl.ds(base + k * chunk, chunk)],
                            idx_v.at[k])
        for k in range(nch):
            pltpu.sync_copy(x_hbm.at[pl.ds(base + k * chunk, chunk)], xbuf)
            pltpu.sync_copy(w16_hbm.at[pl.ds(base + k * chunk, chunk)], wbuf)
            cpx = pltpu.async_copy(xbuf, xs_hbm.at[idx_v.at[k]], semx)
            cpw = pltpu.async_copy(wbuf, ws_hbm.at[idx_v.at[k]], semw)
            cpx.wait()
            cpw.wait()

    return dispatch


def _gmm_body(te_ref, xs_ref, ws_ref, we_ref, be_ref, ys_ref):
    v = te_ref[pl.program_id(0)]

    @pl.when(v < 8)
    def _():
        e = v
        we = we_ref[pl.ds(e, 1), :, :][0]
        acc = lax.dot_general(xs_ref[...], we, (((1,), (1,)), ((), ())),
                              preferred_element_type=jnp.float32)
        ys_ref[...] = (acc + be_ref[pl.ds(e, 1), 0, :]) * ws_ref[:, 0:1]


def _make_collect(n, pad_n, d, nw, chunk):
    rows_w = n // nw
    nch = rows_w // chunk
    mesh = plsc.VectorSubcoreMesh(core_axis_name="c", subcore_axis_name="s")

    @functools.partial(
        pl.kernel, mesh=mesh,
        out_type=jax.ShapeDtypeStruct((n, d), jnp.float32),
        scratch_types=[pltpu.VMEM((rows_w,), jnp.int32),
                       pltpu.VMEM((chunk, d), jnp.float32),
                       pltpu.SemaphoreType.DMA],
    )
    def collect(ys_hbm, pos_hbm, out_hbm, idx_v, buf, sem):
        cid = lax.axis_index("c")
        sid = lax.axis_index("s")
        wid = sid * 2 + cid
        base = wid * rows_w
        pltpu.sync_copy(pos_hbm.at[pl.ds(base, rows_w)], idx_v)
        for k in range(nch):
            cp = pltpu.async_copy(
                ys_hbm.at[idx_v.at[pl.ds(k * chunk, chunk)]], buf, sem)
            cp.wait()
            pltpu.sync_copy(buf, out_hbm.at[pl.ds(base + k * chunk, chunk)])

    return collect


def _gmm(te_flat, xs, ws, We, be3, g, tm, d):
    grid_spec = pltpu.PrefetchScalarGridSpec(
        num_scalar_prefetch=1,
        grid=(g,),
        in_specs=[
            pl.BlockSpec((tm, d), lambda g, te: (g, 0)),
            pl.BlockSpec((tm, 128), lambda g, te: (g, 0)),
            pl.BlockSpec((8, d, d), lambda g, te: (0, 0, 0)),
            pl.BlockSpec((8, 1, d), lambda g, te: (0, 0, 0)),
        ],
        out_specs=pl.BlockSpec((tm, d), lambda g, te: (g, 0)),
    )
    return pl.pallas_call(
        _gmm_body,
        grid_spec=grid_spec,
        out_shape=jax.ShapeDtypeStruct((xs.shape[0], d), jnp.float32),
        compiler_params=pltpu.CompilerParams(
            dimension_semantics=("arbitrary",)),
    )(te_flat, xs, ws, We, be3)


def kernel(x, Wr, br, We, be):
    B, S, D = x.shape
    E = Wr.shape[0]
    N = B * S
    PAD_N = N + E * _TM
    G = PAD_N // _TM
    NW = 32

    x2 = x.reshape(N, D)
    br2 = br.reshape(1, E)
    be3 = be.reshape(E, 1, D)

    w16, pos64, te8 = pl.pallas_call(
        functools.partial(_route_body, num_e=E, tm=_TM),
        grid=(N // _TILE_R,),
        in_specs=[
            pl.BlockSpec((_TILE_R, D), lambda m: (m, 0)),
            pl.BlockSpec((E, D), lambda m: (0, 0)),
            pl.BlockSpec((1, E), lambda m: (0, 0)),
        ],
        out_specs=[
            pl.BlockSpec((_TILE_R, 128), lambda m: (m, 0)),
            pl.BlockSpec((N // 128, 128), lambda m: (0, 0)),
            pl.BlockSpec((8, 128), lambda m: (0, 0)),
        ],
        out_shape=[
            jax.ShapeDtypeStruct((N, 128), jnp.float32),
            jax.ShapeDtypeStruct((N // 128, 128), jnp.int32),
            jax.ShapeDtypeStruct((8, 128), jnp.int32),
        ],
        scratch_shapes=[pltpu.VMEM((N // 128, 128), jnp.int32)],
        compiler_params=pltpu.CompilerParams(
            dimension_semantics=("arbitrary",)),
    )(x2, Wr, br2)
    pos_flat = pos64.reshape(N)
    te_flat = te8.reshape(-1)[:G]

    xs, ws = _make_dispatch(N, PAD_N, D, NW, 64)(x2, w16, pos_flat)
    ys = _gmm(te_flat, xs, ws, We, be3, G, _TM, D)
    out2 = _make_collect(N, PAD_N, D, NW, 64)(ys, pos_flat)
    return out2.reshape(B, S, D)


# speedup vs baseline: 1.4940x; 1.0067x over previous
import functools

import jax
import jax.numpy as jnp
from jax import lax
from jax.experimental import pallas as pl
from jax.experimental.pallas import tpu as pltpu
from jax.experimental.pallas import tpu_sc as plsc

_TM = 512
_TILE_R = 2048


def _route_body(x_ref, wr_ref, br_ref, w16_ref, pos_ref, te_ref, est_scr,
                num_e, tm):
    m_id = pl.program_id(0)
    xt = x_ref[...]
    logits = lax.dot_general(xt, wr_ref[...], (((1,), (1,)), ((), ())),
                             preferred_element_type=jnp.float32)
    logits = logits + br_ref[...]
    max1 = jnp.max(logits, axis=1, keepdims=True)
    ex = jnp.exp(logits - max1)
    probs = ex / jnp.sum(ex, axis=1, keepdims=True)
    iota_e = lax.broadcasted_iota(jnp.int32, logits.shape, 1)
    big = jnp.asarray(logits.shape[1], jnp.int32)
    e1 = jnp.min(jnp.where(logits == max1, iota_e, big), axis=1, keepdims=True)
    l2 = jnp.where(iota_e == e1, -jnp.inf, logits)
    max2 = jnp.max(l2, axis=1, keepdims=True)
    e2 = jnp.min(jnp.where(l2 == max2, iota_e, big), axis=1, keepdims=True)
    es = jnp.maximum(e1, e2)
    w = jnp.sum(jnp.where(iota_e == es, probs, 0.0), axis=1, keepdims=True)
    rows_t = xt.shape[0] // 128
    est_scr[pl.ds(m_id * rows_t, rows_t), :] = es.reshape(rows_t, 128)
    w16_ref[...] = jnp.broadcast_to(w, (w.shape[0], 128))

    @pl.when(m_id == pl.num_programs(0) - 1)
    def _sort():
        pos, te = _sort_half(est_scr[...], num_e, tm)
        pos_ref[...] = pos
        te_ref[...] = te


def _rank_flat(m, lane, sub):
    rows, lanes = m.shape
    c = m
    k = 1
    while k < lanes:
        c = c + jnp.where(lane >= k, pltpu.roll(c, k, 1), 0)
        k *= 2
    rowtot = c[:, lanes - 1:lanes]
    r = rowtot
    k = 1
    while k < rows:
        r = r + jnp.where(sub >= k, pltpu.roll(r, k, 0), 0)
        k *= 2
    return (c - m) + (r - rowtot)


def _sort_half(est, num_e, tm):
    rows, lanes = est.shape
    lane = lax.broadcasted_iota(jnp.int32, (rows, lanes), 1)
    sub = lax.broadcasted_iota(jnp.int32, (rows, 1), 0)
    pos = jnp.zeros((rows, lanes), jnp.int32)
    offs = []
    cnts = []
    off = jnp.asarray(0, jnp.int32)
    for e in range(num_e):
        m = (est == e).astype(jnp.int32)
        excl = _rank_flat(m, lane, sub)
        pos = pos + jnp.where(m, excl + off, 0)
        offs.append(off)
        cnt = jnp.sum(m)
        cnts.append(cnt)
        off = off + ((cnt + (tm - 1)) // tm) * tm
    bi = (lax.broadcasted_iota(jnp.int32, (8, 128), 0) * 128
          + lax.broadcasted_iota(jnp.int32, (8, 128), 1))
    te = jnp.zeros((8, 128), jnp.int32)
    act = jnp.zeros((8, 128), jnp.int32)
    for e in range(num_e):
        te = te + jnp.where(bi * tm >= offs[e], 1, 0)
        act = act | ((bi * tm >= offs[e]) & (bi * tm < offs[e] + cnts[e]))
    return pos, (te - 1) + 8 * (1 - act)


def _make_dispatch(n, pad_n, d, nw, chunk):
    rows_w = n // nw
    nch = rows_w // chunk
    mesh = plsc.VectorSubcoreMesh(core_axis_name="c", subcore_axis_name="s")

    @functools.partial(
        pl.kernel, mesh=mesh,
        out_type=(jax.ShapeDtypeStruct((pad_n, d), jnp.float32),
                  jax.ShapeDtypeStruct((pad_n, 128), jnp.float32)),
        scratch_types=[pltpu.VMEM((nch, chunk), jnp.int32),
                       pltpu.VMEM((2, rows_w // 2), jnp.int32),
                       pltpu.VMEM((chunk, d), jnp.float32),
                       pltpu.VMEM((rows_w, 128), jnp.float32),
                       pltpu.SemaphoreType.DMA,
                       pltpu.SemaphoreType.DMA],
    )
    def dispatch(x_hbm, w16_hbm, pos_hbm, xs_hbm, ws_hbm,
                 idx_v, widx_v, xbuf, wbuf, semx, semw):
        cid = lax.axis_index("c")
        sid = lax.axis_index("s")
        wid = sid * 2 + cid
        base = wid * rows_w
        half = rows_w // 2
        for k in range(nch):
            pltpu.sync_copy(pos_hbm.at[pl.ds(base + k * chunk, chunk)],
                            idx_v.at[k])
        for k in range(2):
            pltpu.sync_copy(pos_hbm.at[pl.ds(base + k * half, half)],
                            widx_v.at[k])
        pltpu.sync_copy(w16_hbm.at[pl.ds(base, rows_w)], wbuf)
        cpw0 = pltpu.async_copy(wbuf.at[pl.ds(0, half)],
                                ws_hbm.at[widx_v.at[0]], semw)
        cpw1 = pltpu.async_copy(wbuf.at[pl.ds(half, half)],
                                ws_hbm.at[widx_v.at[1]], semw)
        for k in range(nch):
            pltpu.sync_copy(x_hbm.at[pl.ds(base + k * chunk, chunk)], xbuf)
            cpx = pltpu.async_copy(xbuf, xs_hbm.at[idx_v.at[k]], semx)
            cpx.wait()
        cpw0.wait()
        cpw1.wait()

    return dispatch


def _gmm_body(te_ref, xs_ref, ws_ref, we_ref, be_ref, ys_ref):
    v = te_ref[pl.program_id(0)]

    @pl.when(v < 8)
    def _():
        e = v
        we = we_ref[pl.ds(e, 1), :, :][0]
        acc = lax.dot_general(xs_ref[...], we, (((1,), (1,)), ((), ())),
                              preferred_element_type=jnp.float32)
        ys_ref[...] = (acc + be_ref[pl.ds(e, 1), 0, :]) * ws_ref[:, 0:1]


def _make_collect(n, pad_n, d, nw, chunk):
    rows_w = n // nw
    nch = rows_w // chunk
    mesh = plsc.VectorSubcoreMesh(core_axis_name="c", subcore_axis_name="s")

    @functools.partial(
        pl.kernel, mesh=mesh,
        out_type=jax.ShapeDtypeStruct((n, d), jnp.float32),
        scratch_types=[pltpu.VMEM((rows_w,), jnp.int32),
                       pltpu.VMEM((chunk, d), jnp.float32),
                       pltpu.SemaphoreType.DMA],
    )
    def collect(ys_hbm, pos_hbm, out_hbm, idx_v, buf, sem):
        cid = lax.axis_index("c")
        sid = lax.axis_index("s")
        wid = sid * 2 + cid
        base = wid * rows_w
        pltpu.sync_copy(pos_hbm.at[pl.ds(base, rows_w)], idx_v)
        for k in range(nch):
            cp = pltpu.async_copy(
                ys_hbm.at[idx_v.at[pl.ds(k * chunk, chunk)]], buf, sem)
            cp.wait()
            pltpu.sync_copy(buf, out_hbm.at[pl.ds(base + k * chunk, chunk)])

    return collect


def _gmm(te_flat, xs, ws, We, be3, g, tm, d):
    grid_spec = pltpu.PrefetchScalarGridSpec(
        num_scalar_prefetch=1,
        grid=(g,),
        in_specs=[
            pl.BlockSpec((tm, d), lambda g, te: (g, 0)),
            pl.BlockSpec((tm, 128), lambda g, te: (g, 0)),
            pl.BlockSpec((8, d, d), lambda g, te: (0, 0, 0)),
            pl.BlockSpec((8, 1, d), lambda g, te: (0, 0, 0)),
        ],
        out_specs=pl.BlockSpec((tm, d), lambda g, te: (g, 0)),
    )
    return pl.pallas_call(
        _gmm_body,
        grid_spec=grid_spec,
        out_shape=jax.ShapeDtypeStruct((xs.shape[0], d), jnp.float32),
        compiler_params=pltpu.CompilerParams(
            dimension_semantics=("arbitrary",)),
    )(te_flat, xs, ws, We, be3)


def kernel(x, Wr, br, We, be):
    B, S, D = x.shape
    E = Wr.shape[0]
    N = B * S
    PAD_N = N + E * _TM
    G = PAD_N // _TM
    NW = 32

    x2 = x.reshape(N, D)
    br2 = br.reshape(1, E)
    be3 = be.reshape(E, 1, D)

    w16, pos64, te8 = pl.pallas_call(
        functools.partial(_route_body, num_e=E, tm=_TM),
        grid=(N // _TILE_R,),
        in_specs=[
            pl.BlockSpec((_TILE_R, D), lambda m: (m, 0)),
            pl.BlockSpec((E, D), lambda m: (0, 0)),
            pl.BlockSpec((1, E), lambda m: (0, 0)),
        ],
        out_specs=[
            pl.BlockSpec((_TILE_R, 128), lambda m: (m, 0)),
            pl.BlockSpec((N // 128, 128), lambda m: (0, 0)),
            pl.BlockSpec((8, 128), lambda m: (0, 0)),
        ],
        out_shape=[
            jax.ShapeDtypeStruct((N, 128), jnp.float32),
            jax.ShapeDtypeStruct((N // 128, 128), jnp.int32),
            jax.ShapeDtypeStruct((8, 128), jnp.int32),
        ],
        scratch_shapes=[pltpu.VMEM((N // 128, 128), jnp.int32)],
        compiler_params=pltpu.CompilerParams(
            dimension_semantics=("arbitrary",)),
    )(x2, Wr, br2)
    pos_flat = pos64.reshape(N)
    te_flat = te8.reshape(-1)[:G]

    xs, ws = _make_dispatch(N, PAD_N, D, NW, 64)(x2, w16, pos_flat)
    ys = _gmm(te_flat, xs, ws, We, be3, G, _TM, D)
    out2 = _make_collect(N, PAD_N, D, NW, 64)(ys, pos_flat)
    return out2.reshape(B, S, D)
